# baseline (device time: 33083 ns/iter reference)
import os

import jax
import jax.numpy as jnp
from jax import lax
from jax.experimental import pallas as pl
from jax.experimental.pallas import tpu as pltpu

N_CHUNK = 4
_vfile = os.path.join(os.path.dirname(os.path.abspath(__file__)), "variant.txt")
try:
    with open(_vfile) as _f:
        _VARIANT = _f.read().strip() or "full"
except OSError:
    _VARIANT = "full"


def _kernel_compute_only(A, B):
    m, k = A.shape
    _, n = B.shape

    def body(a_ref, b_ref, out_ref):
        out_ref[...] = jnp.dot(
            a_ref[...], b_ref[...], preferred_element_type=jnp.float32
        )

    return pl.pallas_call(
        body,
        out_shape=jax.ShapeDtypeStruct((m, n), jnp.float32),
        in_specs=[pl.BlockSpec(memory_space=pltpu.VMEM)] * 2,
        out_specs=pl.BlockSpec(memory_space=pltpu.VMEM),
    )(A, B)


def _kernel_rdma_only(A, B):
    m, k = A.shape
    _, n = B.shape

    def body(a_ref, b_ref, out_ref, comm_ref, send_sem, recv_sem):
        my_x = lax.axis_index("x")
        my_y = lax.axis_index("y")
        nbr = (1 - my_x, my_y)
        barrier_sem = pltpu.get_barrier_semaphore()
        pl.semaphore_signal(barrier_sem, inc=1, device_id=nbr,
                            device_id_type=pl.DeviceIdType.MESH)
        pl.semaphore_wait(barrier_sem, 1)
        out_ref[...] = jnp.zeros_like(out_ref)
        rdma = pltpu.make_async_remote_copy(
            src_ref=out_ref, dst_ref=comm_ref,
            send_sem=send_sem, recv_sem=recv_sem,
            device_id=nbr, device_id_type=pl.DeviceIdType.MESH)
        rdma.start()
        rdma.wait()
        out_ref[0:8, :] = comm_ref[0:8, :]

    return pl.pallas_call(
        body,
        out_shape=jax.ShapeDtypeStruct((m, n), jnp.float32),
        in_specs=[pl.BlockSpec(memory_space=pltpu.VMEM)] * 2,
        out_specs=pl.BlockSpec(memory_space=pltpu.VMEM),
        scratch_shapes=[
            pltpu.VMEM((m, n), jnp.float32),
            pltpu.SemaphoreType.DMA,
            pltpu.SemaphoreType.DMA,
        ],
        compiler_params=pltpu.CompilerParams(collective_id=0),
    )(A, B)


def _kernel_full(A, B):
    m, k = A.shape
    _, n = B.shape
    mc = m // N_CHUNK

    def body(a_ref, b_ref, out_ref, comm_ref, send_sems, recv_sems):
        my_x = lax.axis_index("x")
        my_y = lax.axis_index("y")
        nbr = (1 - my_x, my_y)

        barrier_sem = pltpu.get_barrier_semaphore()
        pl.semaphore_signal(
            barrier_sem, inc=1, device_id=nbr,
            device_id_type=pl.DeviceIdType.MESH,
        )
        pl.semaphore_wait(barrier_sem, 1)

        rdmas = []
        for i in range(N_CHUNK):
            sl = pl.ds(i * mc, mc)
            out_ref[sl, :] = jnp.dot(
                a_ref[sl, :], b_ref[...], preferred_element_type=jnp.float32
            )
            rdma = pltpu.make_async_remote_copy(
                src_ref=out_ref.at[sl, :],
                dst_ref=comm_ref.at[i],
                send_sem=send_sems.at[i],
                recv_sem=recv_sems.at[i],
                device_id=nbr,
                device_id_type=pl.DeviceIdType.MESH,
            )
            rdma.start()
            rdmas.append(rdma)

        for i in range(N_CHUNK):
            sl = pl.ds(i * mc, mc)
            rdmas[i].wait_send()
            rdmas[i].wait_recv()
            out_ref[sl, :] = out_ref[sl, :] + comm_ref[i]

    return pl.pallas_call(
        body,
        out_shape=jax.ShapeDtypeStruct((m, n), jnp.float32),
        in_specs=[
            pl.BlockSpec(memory_space=pltpu.VMEM),
            pl.BlockSpec(memory_space=pltpu.VMEM),
        ],
        out_specs=pl.BlockSpec(memory_space=pltpu.VMEM),
        scratch_shapes=[
            pltpu.VMEM((N_CHUNK, mc, n), jnp.float32),
            pltpu.SemaphoreType.DMA((N_CHUNK,)),
            pltpu.SemaphoreType.DMA((N_CHUNK,)),
        ],
        compiler_params=pltpu.CompilerParams(collective_id=0),
    )(A, B)


kernel = {
    "compute_only": _kernel_compute_only,
    "rdma_only": _kernel_rdma_only,
    "full": _kernel_full,
}[_VARIANT]


# device time: 28372 ns/iter; 1.1660x vs baseline; 1.1660x over previous
import os

import jax
import jax.numpy as jnp
from jax import lax
from jax.experimental import pallas as pl
from jax.experimental.pallas import tpu as pltpu

N_CHUNK = 4
_vfile = os.path.join(os.path.dirname(os.path.abspath(__file__)), "variant.txt")
try:
    with open(_vfile) as _f:
        _VARIANT = _f.read().strip() or "full"
except OSError:
    _VARIANT = "full"


def _kernel_compute_only(A, B):
    m, k = A.shape
    _, n = B.shape

    def body(a_ref, b_ref, out_ref):
        out_ref[...] = jnp.dot(
            a_ref[...], b_ref[...], preferred_element_type=jnp.float32
        )

    return pl.pallas_call(
        body,
        out_shape=jax.ShapeDtypeStruct((m, n), jnp.float32),
        in_specs=[pl.BlockSpec(memory_space=pltpu.VMEM)] * 2,
        out_specs=pl.BlockSpec(memory_space=pltpu.VMEM),
    )(A, B)


def _kernel_rdma_only(A, B):
    m, k = A.shape
    _, n = B.shape

    def body(a_ref, b_ref, out_ref, comm_ref, send_sem, recv_sem):
        my_x = lax.axis_index("x")
        my_y = lax.axis_index("y")
        nbr = (1 - my_x, my_y)
        barrier_sem = pltpu.get_barrier_semaphore()
        pl.semaphore_signal(barrier_sem, inc=1, device_id=nbr,
                            device_id_type=pl.DeviceIdType.MESH)
        pl.semaphore_wait(barrier_sem, 1)
        out_ref[...] = jnp.zeros_like(out_ref)
        rdma = pltpu.make_async_remote_copy(
            src_ref=out_ref, dst_ref=comm_ref,
            send_sem=send_sem, recv_sem=recv_sem,
            device_id=nbr, device_id_type=pl.DeviceIdType.MESH)
        rdma.start()
        rdma.wait()
        out_ref[0:8, :] = comm_ref[0:8, :]

    return pl.pallas_call(
        body,
        out_shape=jax.ShapeDtypeStruct((m, n), jnp.float32),
        in_specs=[pl.BlockSpec(memory_space=pltpu.VMEM)] * 2,
        out_specs=pl.BlockSpec(memory_space=pltpu.VMEM),
        scratch_shapes=[
            pltpu.VMEM((m, n), jnp.float32),
            pltpu.SemaphoreType.DMA,
            pltpu.SemaphoreType.DMA,
        ],
        compiler_params=pltpu.CompilerParams(collective_id=0),
    )(A, B)


def _kernel_full(A, B):
    m, k = A.shape
    _, n = B.shape
    mc = m // N_CHUNK

    def body(a_ref, b_ref, out_ref, comm_ref, send_sems, recv_sems):
        my_x = lax.axis_index("x")
        my_y = lax.axis_index("y")
        nbr = (1 - my_x, my_y)

        barrier_sem = pltpu.get_barrier_semaphore()
        pl.semaphore_signal(
            barrier_sem, inc=1, device_id=nbr,
            device_id_type=pl.DeviceIdType.MESH,
        )
        pl.semaphore_wait(barrier_sem, 1)

        rdmas = []
        for i in range(N_CHUNK):
            sl = pl.ds(i * mc, mc)
            out_ref[sl, :] = jnp.dot(
                a_ref[sl, :], b_ref[...], preferred_element_type=jnp.float32
            )
            rdma = pltpu.make_async_remote_copy(
                src_ref=out_ref.at[sl, :],
                dst_ref=comm_ref.at[i],
                send_sem=send_sems.at[i],
                recv_sem=recv_sems.at[i],
                device_id=nbr,
                device_id_type=pl.DeviceIdType.MESH,
            )
            rdma.start()
            rdmas.append(rdma)

        for i in range(N_CHUNK):
            sl = pl.ds(i * mc, mc)
            rdmas[i].wait_send()
            rdmas[i].wait_recv()
            out_ref[sl, :] = out_ref[sl, :] + comm_ref[i]

    return pl.pallas_call(
        body,
        out_shape=jax.ShapeDtypeStruct((m, n), jnp.float32),
        in_specs=[
            pl.BlockSpec(memory_space=pltpu.VMEM),
            pl.BlockSpec(memory_space=pltpu.VMEM),
        ],
        out_specs=pl.BlockSpec(memory_space=pltpu.VMEM),
        scratch_shapes=[
            pltpu.VMEM((N_CHUNK, mc, n), jnp.float32),
            pltpu.SemaphoreType.DMA((N_CHUNK,)),
            pltpu.SemaphoreType.DMA((N_CHUNK,)),
        ],
        compiler_params=pltpu.CompilerParams(collective_id=0),
    )(A, B)


def _kernel_v3(A, B):
    m, k = A.shape
    _, n = B.shape
    half = m // 2
    qm = m // 4
    NB = 3
    nc = n // NB

    def body(a_ref, b_ref, out_ref, ar_ref, br_ref,
             a_send, a_recv, af_send, af_recv, b_send, b_recv):
        my_x = lax.axis_index("x")
        my_y = lax.axis_index("y")
        p = (1 - my_x, my_y)
        q = (my_x, 1 - my_y)

        barrier_sem = pltpu.get_barrier_semaphore()
        for nbr in (p, q):
            pl.semaphore_signal(barrier_sem, inc=1, device_id=nbr,
                                device_id_type=pl.DeviceIdType.MESH)
        pl.semaphore_wait(barrier_sem, 2)

        def a_direct_rdma(row0, c):
            sl = slice(row0 + c * qm, row0 + (c + 1) * qm)
            return pltpu.make_async_remote_copy(
                src_ref=a_ref.at[sl, :], dst_ref=ar_ref.at[sl, :],
                send_sem=a_send.at[c], recv_sem=a_recv.at[c],
                device_id=p, device_id_type=pl.DeviceIdType.MESH)

        def a_fwd_rdma(row0, c):
            sl = slice(row0 + c * qm, row0 + (c + 1) * qm)
            return pltpu.make_async_remote_copy(
                src_ref=ar_ref.at[sl, :], dst_ref=ar_ref.at[sl, :],
                send_sem=af_send.at[c], recv_sem=af_recv.at[c],
                device_id=q, device_id_type=pl.DeviceIdType.MESH)

        @pl.when(my_y == 0)
        def _():
            for c in range(2):
                a_direct_rdma(0, c).start()

        @pl.when(my_y == 1)
        def _():
            for c in range(2):
                a_direct_rdma(half, c).start()

        b_rdmas = []
        for j in range(NB):
            slc = slice(j * nc, (j + 1) * nc)
            r = pltpu.make_async_remote_copy(
                src_ref=b_ref.at[:, slc], dst_ref=br_ref.at[:, slc],
                send_sem=b_send.at[j], recv_sem=b_recv.at[j],
                device_id=p, device_id_type=pl.DeviceIdType.MESH)
            r.start()
            b_rdmas.append(r)

        out_ref[...] = jnp.dot(a_ref[...], b_ref[...],
                               preferred_element_type=jnp.float32)

        for c in range(2):
            a_direct_rdma(0, c).wait_recv()

            @pl.when(my_y == 0)
            def _(c=c):
                a_fwd_rdma(0, c).start()

            @pl.when(my_y == 1)
            def _(c=c):
                a_fwd_rdma(half, c).start()

        for c in range(2):
            a_fwd_rdma(0, c).wait_recv()

        for j in range(NB):
            slc = slice(j * nc, (j + 1) * nc)
            b_rdmas[j].wait_recv()
            out_ref[:, slc] = out_ref[:, slc] + jnp.dot(
                ar_ref[...], br_ref[:, slc],
                preferred_element_type=jnp.float32)

        for c in range(2):
            a_direct_rdma(0, c).wait_send()
            a_fwd_rdma(0, c).wait_send()
        for r in b_rdmas:
            r.wait_send()

    return pl.pallas_call(
        body,
        out_shape=jax.ShapeDtypeStruct((m, n), jnp.float32),
        in_specs=[
            pl.BlockSpec(memory_space=pltpu.VMEM),
            pl.BlockSpec(memory_space=pltpu.VMEM),
        ],
        out_specs=pl.BlockSpec(memory_space=pltpu.VMEM),
        scratch_shapes=[
            pltpu.VMEM((m, k), jnp.float32),
            pltpu.VMEM((k, n), jnp.float32),
            pltpu.SemaphoreType.DMA((2,)),
            pltpu.SemaphoreType.DMA((2,)),
            pltpu.SemaphoreType.DMA((2,)),
            pltpu.SemaphoreType.DMA((2,)),
            pltpu.SemaphoreType.DMA((NB,)),
            pltpu.SemaphoreType.DMA((NB,)),
        ],
        compiler_params=pltpu.CompilerParams(collective_id=0),
    )(A, B)


kernel = {
    "compute_only": _kernel_compute_only,
    "rdma_only": _kernel_rdma_only,
    "full": _kernel_full,
    "v3": _kernel_v3,
}[_VARIANT]


# device time: 19198 ns/iter; 1.7233x vs baseline; 1.4779x over previous
import os

import jax
import jax.numpy as jnp
from jax import lax
from jax.experimental import pallas as pl
from jax.experimental.pallas import tpu as pltpu

N_CHUNK = 4
_vfile = os.path.join(os.path.dirname(os.path.abspath(__file__)), "variant.txt")
try:
    with open(_vfile) as _f:
        _VARIANT = _f.read().strip() or "full"
except OSError:
    _VARIANT = "full"


def _kernel_compute_only(A, B):
    m, k = A.shape
    _, n = B.shape

    def body(a_ref, b_ref, out_ref):
        out_ref[...] = jnp.dot(
            a_ref[...], b_ref[...], preferred_element_type=jnp.float32
        )

    return pl.pallas_call(
        body,
        out_shape=jax.ShapeDtypeStruct((m, n), jnp.float32),
        in_specs=[pl.BlockSpec(memory_space=pltpu.VMEM)] * 2,
        out_specs=pl.BlockSpec(memory_space=pltpu.VMEM),
    )(A, B)


def _kernel_rdma_only(A, B):
    m, k = A.shape
    _, n = B.shape

    def body(a_ref, b_ref, out_ref, comm_ref, send_sem, recv_sem):
        my_x = lax.axis_index("x")
        my_y = lax.axis_index("y")
        nbr = (1 - my_x, my_y)
        barrier_sem = pltpu.get_barrier_semaphore()
        pl.semaphore_signal(barrier_sem, inc=1, device_id=nbr,
                            device_id_type=pl.DeviceIdType.MESH)
        pl.semaphore_wait(barrier_sem, 1)
        out_ref[...] = jnp.zeros_like(out_ref)
        rdma = pltpu.make_async_remote_copy(
            src_ref=out_ref, dst_ref=comm_ref,
            send_sem=send_sem, recv_sem=recv_sem,
            device_id=nbr, device_id_type=pl.DeviceIdType.MESH)
        rdma.start()
        rdma.wait()
        out_ref[0:8, :] = comm_ref[0:8, :]

    return pl.pallas_call(
        body,
        out_shape=jax.ShapeDtypeStruct((m, n), jnp.float32),
        in_specs=[pl.BlockSpec(memory_space=pltpu.VMEM)] * 2,
        out_specs=pl.BlockSpec(memory_space=pltpu.VMEM),
        scratch_shapes=[
            pltpu.VMEM((m, n), jnp.float32),
            pltpu.SemaphoreType.DMA,
            pltpu.SemaphoreType.DMA,
        ],
        compiler_params=pltpu.CompilerParams(collective_id=0),
    )(A, B)


def _kernel_full(A, B):
    m, k = A.shape
    _, n = B.shape
    mc = m // N_CHUNK

    def body(a_ref, b_ref, out_ref, comm_ref, send_sems, recv_sems):
        my_x = lax.axis_index("x")
        my_y = lax.axis_index("y")
        nbr = (1 - my_x, my_y)

        barrier_sem = pltpu.get_barrier_semaphore()
        pl.semaphore_signal(
            barrier_sem, inc=1, device_id=nbr,
            device_id_type=pl.DeviceIdType.MESH,
        )
        pl.semaphore_wait(barrier_sem, 1)

        rdmas = []
        for i in range(N_CHUNK):
            sl = pl.ds(i * mc, mc)
            out_ref[sl, :] = jnp.dot(
                a_ref[sl, :], b_ref[...], preferred_element_type=jnp.float32
            )
            rdma = pltpu.make_async_remote_copy(
                src_ref=out_ref.at[sl, :],
                dst_ref=comm_ref.at[i],
                send_sem=send_sems.at[i],
                recv_sem=recv_sems.at[i],
                device_id=nbr,
                device_id_type=pl.DeviceIdType.MESH,
            )
            rdma.start()
            rdmas.append(rdma)

        for i in range(N_CHUNK):
            sl = pl.ds(i * mc, mc)
            rdmas[i].wait_send()
            rdmas[i].wait_recv()
            out_ref[sl, :] = out_ref[sl, :] + comm_ref[i]

    return pl.pallas_call(
        body,
        out_shape=jax.ShapeDtypeStruct((m, n), jnp.float32),
        in_specs=[
            pl.BlockSpec(memory_space=pltpu.VMEM),
            pl.BlockSpec(memory_space=pltpu.VMEM),
        ],
        out_specs=pl.BlockSpec(memory_space=pltpu.VMEM),
        scratch_shapes=[
            pltpu.VMEM((N_CHUNK, mc, n), jnp.float32),
            pltpu.SemaphoreType.DMA((N_CHUNK,)),
            pltpu.SemaphoreType.DMA((N_CHUNK,)),
        ],
        compiler_params=pltpu.CompilerParams(collective_id=0),
    )(A, B)


def _kernel_v3(A, B):
    m, k = A.shape
    _, n = B.shape
    half = m // 2
    qm = m // 4
    NB = 3
    nc = n // NB

    def body(a_ref, b_ref, out_ref, ar_ref, br_ref,
             a_send, a_recv, af_send, af_recv, b_send, b_recv):
        my_x = lax.axis_index("x")
        my_y = lax.axis_index("y")
        p = (1 - my_x, my_y)
        q = (my_x, 1 - my_y)

        barrier_sem = pltpu.get_barrier_semaphore()
        for nbr in (p, q):
            pl.semaphore_signal(barrier_sem, inc=1, device_id=nbr,
                                device_id_type=pl.DeviceIdType.MESH)
        pl.semaphore_wait(barrier_sem, 2)

        def a_direct_rdma(row0, c):
            sl = slice(row0 + c * qm, row0 + (c + 1) * qm)
            return pltpu.make_async_remote_copy(
                src_ref=a_ref.at[sl, :], dst_ref=ar_ref.at[sl, :],
                send_sem=a_send.at[c], recv_sem=a_recv.at[c],
                device_id=p, device_id_type=pl.DeviceIdType.MESH)

        def a_fwd_rdma(row0, c):
            sl = slice(row0 + c * qm, row0 + (c + 1) * qm)
            return pltpu.make_async_remote_copy(
                src_ref=ar_ref.at[sl, :], dst_ref=ar_ref.at[sl, :],
                send_sem=af_send.at[c], recv_sem=af_recv.at[c],
                device_id=q, device_id_type=pl.DeviceIdType.MESH)

        @pl.when(my_y == 0)
        def _():
            for c in range(2):
                a_direct_rdma(0, c).start()

        @pl.when(my_y == 1)
        def _():
            for c in range(2):
                a_direct_rdma(half, c).start()

        b_rdmas = []
        for j in range(NB):
            slc = slice(j * nc, (j + 1) * nc)
            r = pltpu.make_async_remote_copy(
                src_ref=b_ref.at[:, slc], dst_ref=br_ref.at[:, slc],
                send_sem=b_send.at[j], recv_sem=b_recv.at[j],
                device_id=p, device_id_type=pl.DeviceIdType.MESH)
            r.start()
            b_rdmas.append(r)

        out_ref[...] = jnp.dot(a_ref[...], b_ref[...],
                               preferred_element_type=jnp.float32)

        for c in range(2):
            a_direct_rdma(0, c).wait_recv()

            @pl.when(my_y == 0)
            def _(c=c):
                a_fwd_rdma(0, c).start()

            @pl.when(my_y == 1)
            def _(c=c):
                a_fwd_rdma(half, c).start()

        for c in range(2):
            a_fwd_rdma(0, c).wait_recv()

        for j in range(NB):
            slc = slice(j * nc, (j + 1) * nc)
            b_rdmas[j].wait_recv()
            out_ref[:, slc] = out_ref[:, slc] + jnp.dot(
                ar_ref[...], br_ref[:, slc],
                preferred_element_type=jnp.float32)

        for c in range(2):
            a_direct_rdma(0, c).wait_send()
            a_fwd_rdma(0, c).wait_send()
        for r in b_rdmas:
            r.wait_send()

    return pl.pallas_call(
        body,
        out_shape=jax.ShapeDtypeStruct((m, n), jnp.float32),
        in_specs=[
            pl.BlockSpec(memory_space=pltpu.VMEM),
            pl.BlockSpec(memory_space=pltpu.VMEM),
        ],
        out_specs=pl.BlockSpec(memory_space=pltpu.VMEM),
        scratch_shapes=[
            pltpu.VMEM((m, k), jnp.float32),
            pltpu.VMEM((k, n), jnp.float32),
            pltpu.SemaphoreType.DMA((2,)),
            pltpu.SemaphoreType.DMA((2,)),
            pltpu.SemaphoreType.DMA((2,)),
            pltpu.SemaphoreType.DMA((2,)),
            pltpu.SemaphoreType.DMA((NB,)),
            pltpu.SemaphoreType.DMA((NB,)),
        ],
        compiler_params=pltpu.CompilerParams(collective_id=0),
    )(A, B)


def _kernel_v4(A, B):
    m, k = A.shape
    _, n = B.shape
    half = m // 2
    qm = m // 4
    NB = 3
    nc = n // NB

    def body(a_ref, b_ref, out_ref, ab_ref, bb_ref, ar_ref, br_ref,
             a_send, a_recv, af_send, af_recv, b_send, b_recv):
        my_x = lax.axis_index("x")
        my_y = lax.axis_index("y")
        p = (1 - my_x, my_y)
        q = (my_x, 1 - my_y)

        ab_ref[...] = a_ref[...].astype(jnp.bfloat16)
        bb_ref[...] = b_ref[...].astype(jnp.bfloat16)

        barrier_sem = pltpu.get_barrier_semaphore()
        for nbr in (p, q):
            pl.semaphore_signal(barrier_sem, inc=1, device_id=nbr,
                                device_id_type=pl.DeviceIdType.MESH)
        pl.semaphore_wait(barrier_sem, 2)

        def a_direct_rdma(row0, c):
            sl = slice(row0 + c * qm, row0 + (c + 1) * qm)
            return pltpu.make_async_remote_copy(
                src_ref=ab_ref.at[sl, :], dst_ref=ar_ref.at[sl, :],
                send_sem=a_send.at[c], recv_sem=a_recv.at[c],
                device_id=p, device_id_type=pl.DeviceIdType.MESH)

        def a_fwd_rdma(row0, c):
            sl = slice(row0 + c * qm, row0 + (c + 1) * qm)
            return pltpu.make_async_remote_copy(
                src_ref=ar_ref.at[sl, :], dst_ref=ar_ref.at[sl, :],
                send_sem=af_send.at[c], recv_sem=af_recv.at[c],
                device_id=q, device_id_type=pl.DeviceIdType.MESH)

        @pl.when(my_y == 0)
        def _():
            for c in range(2):
                a_direct_rdma(0, c).start()

        @pl.when(my_y == 1)
        def _():
            for c in range(2):
                a_direct_rdma(half, c).start()

        b_rdmas = []
        for j in range(NB):
            slc = slice(j * nc, (j + 1) * nc)
            r = pltpu.make_async_remote_copy(
                src_ref=bb_ref.at[:, slc], dst_ref=br_ref.at[:, slc],
                send_sem=b_send.at[j], recv_sem=b_recv.at[j],
                device_id=p, device_id_type=pl.DeviceIdType.MESH)
            r.start()
            b_rdmas.append(r)

        for c in range(2):
            a_direct_rdma(0, c).wait_recv()

            @pl.when(my_y == 0)
            def _(c=c):
                a_fwd_rdma(0, c).start()

            @pl.when(my_y == 1)
            def _(c=c):
                a_fwd_rdma(half, c).start()

        out_ref[...] = jnp.dot(a_ref[...], b_ref[...],
                               preferred_element_type=jnp.float32)

        for c in range(2):
            a_fwd_rdma(0, c).wait_recv()

        for j in range(NB):
            slc = slice(j * nc, (j + 1) * nc)
            b_rdmas[j].wait_recv()
            out_ref[:, slc] = out_ref[:, slc] + jnp.dot(
                ar_ref[...], br_ref[:, slc],
                preferred_element_type=jnp.float32)

        for c in range(2):
            a_direct_rdma(0, c).wait_send()
            a_fwd_rdma(0, c).wait_send()
        for r in b_rdmas:
            r.wait_send()

    return pl.pallas_call(
        body,
        out_shape=jax.ShapeDtypeStruct((m, n), jnp.float32),
        in_specs=[
            pl.BlockSpec(memory_space=pltpu.VMEM),
            pl.BlockSpec(memory_space=pltpu.VMEM),
        ],
        out_specs=pl.BlockSpec(memory_space=pltpu.VMEM),
        scratch_shapes=[
            pltpu.VMEM((m, k), jnp.bfloat16),
            pltpu.VMEM((k, n), jnp.bfloat16),
            pltpu.VMEM((m, k), jnp.bfloat16),
            pltpu.VMEM((k, n), jnp.bfloat16),
            pltpu.SemaphoreType.DMA((2,)),
            pltpu.SemaphoreType.DMA((2,)),
            pltpu.SemaphoreType.DMA((2,)),
            pltpu.SemaphoreType.DMA((2,)),
            pltpu.SemaphoreType.DMA((NB,)),
            pltpu.SemaphoreType.DMA((NB,)),
        ],
        compiler_params=pltpu.CompilerParams(collective_id=0),
    )(A, B)


kernel = {
    "compute_only": _kernel_compute_only,
    "rdma_only": _kernel_rdma_only,
    "full": _kernel_full,
    "v3": _kernel_v3,
    "v4": _kernel_v4,
}[_VARIANT]


# device time: 19089 ns/iter; 1.7331x vs baseline; 1.0057x over previous
import os

import jax
import jax.numpy as jnp
from jax import lax
from jax.experimental import pallas as pl
from jax.experimental.pallas import tpu as pltpu

N_CHUNK = 4
_vfile = os.path.join(os.path.dirname(os.path.abspath(__file__)), "variant.txt")
try:
    with open(_vfile) as _f:
        _VARIANT = _f.read().strip() or "full"
except OSError:
    _VARIANT = "full"


def _kernel_compute_only(A, B):
    m, k = A.shape
    _, n = B.shape

    def body(a_ref, b_ref, out_ref):
        out_ref[...] = jnp.dot(
            a_ref[...], b_ref[...], preferred_element_type=jnp.float32
        )

    return pl.pallas_call(
        body,
        out_shape=jax.ShapeDtypeStruct((m, n), jnp.float32),
        in_specs=[pl.BlockSpec(memory_space=pltpu.VMEM)] * 2,
        out_specs=pl.BlockSpec(memory_space=pltpu.VMEM),
    )(A, B)


def _kernel_rdma_only(A, B):
    m, k = A.shape
    _, n = B.shape

    def body(a_ref, b_ref, out_ref, comm_ref, send_sem, recv_sem):
        my_x = lax.axis_index("x")
        my_y = lax.axis_index("y")
        nbr = (1 - my_x, my_y)
        barrier_sem = pltpu.get_barrier_semaphore()
        pl.semaphore_signal(barrier_sem, inc=1, device_id=nbr,
                            device_id_type=pl.DeviceIdType.MESH)
        pl.semaphore_wait(barrier_sem, 1)
        out_ref[...] = jnp.zeros_like(out_ref)
        rdma = pltpu.make_async_remote_copy(
            src_ref=out_ref, dst_ref=comm_ref,
            send_sem=send_sem, recv_sem=recv_sem,
            device_id=nbr, device_id_type=pl.DeviceIdType.MESH)
        rdma.start()
        rdma.wait()
        out_ref[0:8, :] = comm_ref[0:8, :]

    return pl.pallas_call(
        body,
        out_shape=jax.ShapeDtypeStruct((m, n), jnp.float32),
        in_specs=[pl.BlockSpec(memory_space=pltpu.VMEM)] * 2,
        out_specs=pl.BlockSpec(memory_space=pltpu.VMEM),
        scratch_shapes=[
            pltpu.VMEM((m, n), jnp.float32),
            pltpu.SemaphoreType.DMA,
            pltpu.SemaphoreType.DMA,
        ],
        compiler_params=pltpu.CompilerParams(collective_id=0),
    )(A, B)


def _kernel_full(A, B):
    m, k = A.shape
    _, n = B.shape
    mc = m // N_CHUNK

    def body(a_ref, b_ref, out_ref, comm_ref, send_sems, recv_sems):
        my_x = lax.axis_index("x")
        my_y = lax.axis_index("y")
        nbr = (1 - my_x, my_y)

        barrier_sem = pltpu.get_barrier_semaphore()
        pl.semaphore_signal(
            barrier_sem, inc=1, device_id=nbr,
            device_id_type=pl.DeviceIdType.MESH,
        )
        pl.semaphore_wait(barrier_sem, 1)

        rdmas = []
        for i in range(N_CHUNK):
            sl = pl.ds(i * mc, mc)
            out_ref[sl, :] = jnp.dot(
                a_ref[sl, :], b_ref[...], preferred_element_type=jnp.float32
            )
            rdma = pltpu.make_async_remote_copy(
                src_ref=out_ref.at[sl, :],
                dst_ref=comm_ref.at[i],
                send_sem=send_sems.at[i],
                recv_sem=recv_sems.at[i],
                device_id=nbr,
                device_id_type=pl.DeviceIdType.MESH,
            )
            rdma.start()
            rdmas.append(rdma)

        for i in range(N_CHUNK):
            sl = pl.ds(i * mc, mc)
            rdmas[i].wait_send()
            rdmas[i].wait_recv()
            out_ref[sl, :] = out_ref[sl, :] + comm_ref[i]

    return pl.pallas_call(
        body,
        out_shape=jax.ShapeDtypeStruct((m, n), jnp.float32),
        in_specs=[
            pl.BlockSpec(memory_space=pltpu.VMEM),
            pl.BlockSpec(memory_space=pltpu.VMEM),
        ],
        out_specs=pl.BlockSpec(memory_space=pltpu.VMEM),
        scratch_shapes=[
            pltpu.VMEM((N_CHUNK, mc, n), jnp.float32),
            pltpu.SemaphoreType.DMA((N_CHUNK,)),
            pltpu.SemaphoreType.DMA((N_CHUNK,)),
        ],
        compiler_params=pltpu.CompilerParams(collective_id=0),
    )(A, B)


def _kernel_v3(A, B):
    m, k = A.shape
    _, n = B.shape
    half = m // 2
    qm = m // 4
    NB = 3
    nc = n // NB

    def body(a_ref, b_ref, out_ref, ar_ref, br_ref,
             a_send, a_recv, af_send, af_recv, b_send, b_recv):
        my_x = lax.axis_index("x")
        my_y = lax.axis_index("y")
        p = (1 - my_x, my_y)
        q = (my_x, 1 - my_y)

        barrier_sem = pltpu.get_barrier_semaphore()
        for nbr in (p, q):
            pl.semaphore_signal(barrier_sem, inc=1, device_id=nbr,
                                device_id_type=pl.DeviceIdType.MESH)
        pl.semaphore_wait(barrier_sem, 2)

        def a_direct_rdma(row0, c):
            sl = slice(row0 + c * qm, row0 + (c + 1) * qm)
            return pltpu.make_async_remote_copy(
                src_ref=a_ref.at[sl, :], dst_ref=ar_ref.at[sl, :],
                send_sem=a_send.at[c], recv_sem=a_recv.at[c],
                device_id=p, device_id_type=pl.DeviceIdType.MESH)

        def a_fwd_rdma(row0, c):
            sl = slice(row0 + c * qm, row0 + (c + 1) * qm)
            return pltpu.make_async_remote_copy(
                src_ref=ar_ref.at[sl, :], dst_ref=ar_ref.at[sl, :],
                send_sem=af_send.at[c], recv_sem=af_recv.at[c],
                device_id=q, device_id_type=pl.DeviceIdType.MESH)

        @pl.when(my_y == 0)
        def _():
            for c in range(2):
                a_direct_rdma(0, c).start()

        @pl.when(my_y == 1)
        def _():
            for c in range(2):
                a_direct_rdma(half, c).start()

        b_rdmas = []
        for j in range(NB):
            slc = slice(j * nc, (j + 1) * nc)
            r = pltpu.make_async_remote_copy(
                src_ref=b_ref.at[:, slc], dst_ref=br_ref.at[:, slc],
                send_sem=b_send.at[j], recv_sem=b_recv.at[j],
                device_id=p, device_id_type=pl.DeviceIdType.MESH)
            r.start()
            b_rdmas.append(r)

        out_ref[...] = jnp.dot(a_ref[...], b_ref[...],
                               preferred_element_type=jnp.float32)

        for c in range(2):
            a_direct_rdma(0, c).wait_recv()

            @pl.when(my_y == 0)
            def _(c=c):
                a_fwd_rdma(0, c).start()

            @pl.when(my_y == 1)
            def _(c=c):
                a_fwd_rdma(half, c).start()

        for c in range(2):
            a_fwd_rdma(0, c).wait_recv()

        for j in range(NB):
            slc = slice(j * nc, (j + 1) * nc)
            b_rdmas[j].wait_recv()
            out_ref[:, slc] = out_ref[:, slc] + jnp.dot(
                ar_ref[...], br_ref[:, slc],
                preferred_element_type=jnp.float32)

        for c in range(2):
            a_direct_rdma(0, c).wait_send()
            a_fwd_rdma(0, c).wait_send()
        for r in b_rdmas:
            r.wait_send()

    return pl.pallas_call(
        body,
        out_shape=jax.ShapeDtypeStruct((m, n), jnp.float32),
        in_specs=[
            pl.BlockSpec(memory_space=pltpu.VMEM),
            pl.BlockSpec(memory_space=pltpu.VMEM),
        ],
        out_specs=pl.BlockSpec(memory_space=pltpu.VMEM),
        scratch_shapes=[
            pltpu.VMEM((m, k), jnp.float32),
            pltpu.VMEM((k, n), jnp.float32),
            pltpu.SemaphoreType.DMA((2,)),
            pltpu.SemaphoreType.DMA((2,)),
            pltpu.SemaphoreType.DMA((2,)),
            pltpu.SemaphoreType.DMA((2,)),
            pltpu.SemaphoreType.DMA((NB,)),
            pltpu.SemaphoreType.DMA((NB,)),
        ],
        compiler_params=pltpu.CompilerParams(collective_id=0),
    )(A, B)


def _kernel_v4(A, B, NB=3, local_bf16=False):
    m, k = A.shape
    _, n = B.shape
    half = m // 2
    qm = m // 4
    nc = n // NB

    def body(a_ref, b_ref, out_ref, ab_ref, bb_ref, ar_ref, br_ref,
             a_send, a_recv, af_send, af_recv, b_send, b_recv):
        my_x = lax.axis_index("x")
        my_y = lax.axis_index("y")
        p = (1 - my_x, my_y)
        q = (my_x, 1 - my_y)

        ab_ref[...] = a_ref[...].astype(jnp.bfloat16)
        bb_ref[...] = b_ref[...].astype(jnp.bfloat16)

        barrier_sem = pltpu.get_barrier_semaphore()
        for nbr in (p, q):
            pl.semaphore_signal(barrier_sem, inc=1, device_id=nbr,
                                device_id_type=pl.DeviceIdType.MESH)
        pl.semaphore_wait(barrier_sem, 2)

        def a_direct_rdma(row0, c):
            sl = slice(row0 + c * qm, row0 + (c + 1) * qm)
            return pltpu.make_async_remote_copy(
                src_ref=ab_ref.at[sl, :], dst_ref=ar_ref.at[sl, :],
                send_sem=a_send.at[c], recv_sem=a_recv.at[c],
                device_id=p, device_id_type=pl.DeviceIdType.MESH)

        def a_fwd_rdma(row0, c):
            sl = slice(row0 + c * qm, row0 + (c + 1) * qm)
            return pltpu.make_async_remote_copy(
                src_ref=ar_ref.at[sl, :], dst_ref=ar_ref.at[sl, :],
                send_sem=af_send.at[c], recv_sem=af_recv.at[c],
                device_id=q, device_id_type=pl.DeviceIdType.MESH)

        @pl.when(my_y == 0)
        def _():
            for c in range(2):
                a_direct_rdma(0, c).start()

        @pl.when(my_y == 1)
        def _():
            for c in range(2):
                a_direct_rdma(half, c).start()

        b_rdmas = []
        for j in range(NB):
            slc = slice(j * nc, (j + 1) * nc)
            r = pltpu.make_async_remote_copy(
                src_ref=bb_ref.at[:, slc], dst_ref=br_ref.at[:, slc],
                send_sem=b_send.at[j], recv_sem=b_recv.at[j],
                device_id=p, device_id_type=pl.DeviceIdType.MESH)
            r.start()
            b_rdmas.append(r)

        for c in range(2):
            a_direct_rdma(0, c).wait_recv()

            @pl.when(my_y == 0)
            def _(c=c):
                a_fwd_rdma(0, c).start()

            @pl.when(my_y == 1)
            def _(c=c):
                a_fwd_rdma(half, c).start()

        if local_bf16:
            out_ref[...] = jnp.dot(ab_ref[...], bb_ref[...],
                                   preferred_element_type=jnp.float32)
        else:
            out_ref[...] = jnp.dot(a_ref[...], b_ref[...],
                                   preferred_element_type=jnp.float32)

        for c in range(2):
            a_fwd_rdma(0, c).wait_recv()

        for j in range(NB):
            slc = slice(j * nc, (j + 1) * nc)
            b_rdmas[j].wait_recv()
            out_ref[:, slc] = out_ref[:, slc] + jnp.dot(
                ar_ref[...], br_ref[:, slc],
                preferred_element_type=jnp.float32)

        for c in range(2):
            a_direct_rdma(0, c).wait_send()
            a_fwd_rdma(0, c).wait_send()
        for r in b_rdmas:
            r.wait_send()

    return pl.pallas_call(
        body,
        out_shape=jax.ShapeDtypeStruct((m, n), jnp.float32),
        in_specs=[
            pl.BlockSpec(memory_space=pltpu.VMEM),
            pl.BlockSpec(memory_space=pltpu.VMEM),
        ],
        out_specs=pl.BlockSpec(memory_space=pltpu.VMEM),
        scratch_shapes=[
            pltpu.VMEM((m, k), jnp.bfloat16),
            pltpu.VMEM((k, n), jnp.bfloat16),
            pltpu.VMEM((m, k), jnp.bfloat16),
            pltpu.VMEM((k, n), jnp.bfloat16),
            pltpu.SemaphoreType.DMA((2,)),
            pltpu.SemaphoreType.DMA((2,)),
            pltpu.SemaphoreType.DMA((2,)),
            pltpu.SemaphoreType.DMA((2,)),
            pltpu.SemaphoreType.DMA((NB,)),
            pltpu.SemaphoreType.DMA((NB,)),
        ],
        compiler_params=pltpu.CompilerParams(collective_id=0),
    )(A, B)


def _kernel_v5(A, B):
    return _kernel_v4(A, B, NB=6, local_bf16=True)


kernel = {
    "compute_only": _kernel_compute_only,
    "rdma_only": _kernel_rdma_only,
    "full": _kernel_full,
    "v3": _kernel_v3,
    "v4": _kernel_v4,
    "v5": _kernel_v5,
}[_VARIANT]


# device time: 18606 ns/iter; 1.7781x vs baseline; 1.0260x over previous
import os

import jax
import jax.numpy as jnp
from jax import lax
from jax.experimental import pallas as pl
from jax.experimental.pallas import tpu as pltpu

N_CHUNK = 4
_vfile = os.path.join(os.path.dirname(os.path.abspath(__file__)), "variant.txt")
try:
    with open(_vfile) as _f:
        _VARIANT = _f.read().strip() or "full"
except OSError:
    _VARIANT = "full"


def _kernel_compute_only(A, B):
    m, k = A.shape
    _, n = B.shape

    def body(a_ref, b_ref, out_ref):
        out_ref[...] = jnp.dot(
            a_ref[...], b_ref[...], preferred_element_type=jnp.float32
        )

    return pl.pallas_call(
        body,
        out_shape=jax.ShapeDtypeStruct((m, n), jnp.float32),
        in_specs=[pl.BlockSpec(memory_space=pltpu.VMEM)] * 2,
        out_specs=pl.BlockSpec(memory_space=pltpu.VMEM),
    )(A, B)


def _kernel_rdma_only(A, B):
    m, k = A.shape
    _, n = B.shape

    def body(a_ref, b_ref, out_ref, comm_ref, send_sem, recv_sem):
        my_x = lax.axis_index("x")
        my_y = lax.axis_index("y")
        nbr = (1 - my_x, my_y)
        barrier_sem = pltpu.get_barrier_semaphore()
        pl.semaphore_signal(barrier_sem, inc=1, device_id=nbr,
                            device_id_type=pl.DeviceIdType.MESH)
        pl.semaphore_wait(barrier_sem, 1)
        out_ref[...] = jnp.zeros_like(out_ref)
        rdma = pltpu.make_async_remote_copy(
            src_ref=out_ref, dst_ref=comm_ref,
            send_sem=send_sem, recv_sem=recv_sem,
            device_id=nbr, device_id_type=pl.DeviceIdType.MESH)
        rdma.start()
        rdma.wait()
        out_ref[0:8, :] = comm_ref[0:8, :]

    return pl.pallas_call(
        body,
        out_shape=jax.ShapeDtypeStruct((m, n), jnp.float32),
        in_specs=[pl.BlockSpec(memory_space=pltpu.VMEM)] * 2,
        out_specs=pl.BlockSpec(memory_space=pltpu.VMEM),
        scratch_shapes=[
            pltpu.VMEM((m, n), jnp.float32),
            pltpu.SemaphoreType.DMA,
            pltpu.SemaphoreType.DMA,
        ],
        compiler_params=pltpu.CompilerParams(collective_id=0),
    )(A, B)


def _kernel_full(A, B):
    m, k = A.shape
    _, n = B.shape
    mc = m // N_CHUNK

    def body(a_ref, b_ref, out_ref, comm_ref, send_sems, recv_sems):
        my_x = lax.axis_index("x")
        my_y = lax.axis_index("y")
        nbr = (1 - my_x, my_y)

        barrier_sem = pltpu.get_barrier_semaphore()
        pl.semaphore_signal(
            barrier_sem, inc=1, device_id=nbr,
            device_id_type=pl.DeviceIdType.MESH,
        )
        pl.semaphore_wait(barrier_sem, 1)

        rdmas = []
        for i in range(N_CHUNK):
            sl = pl.ds(i * mc, mc)
            out_ref[sl, :] = jnp.dot(
                a_ref[sl, :], b_ref[...], preferred_element_type=jnp.float32
            )
            rdma = pltpu.make_async_remote_copy(
                src_ref=out_ref.at[sl, :],
                dst_ref=comm_ref.at[i],
                send_sem=send_sems.at[i],
                recv_sem=recv_sems.at[i],
                device_id=nbr,
                device_id_type=pl.DeviceIdType.MESH,
            )
            rdma.start()
            rdmas.append(rdma)

        for i in range(N_CHUNK):
            sl = pl.ds(i * mc, mc)
            rdmas[i].wait_send()
            rdmas[i].wait_recv()
            out_ref[sl, :] = out_ref[sl, :] + comm_ref[i]

    return pl.pallas_call(
        body,
        out_shape=jax.ShapeDtypeStruct((m, n), jnp.float32),
        in_specs=[
            pl.BlockSpec(memory_space=pltpu.VMEM),
            pl.BlockSpec(memory_space=pltpu.VMEM),
        ],
        out_specs=pl.BlockSpec(memory_space=pltpu.VMEM),
        scratch_shapes=[
            pltpu.VMEM((N_CHUNK, mc, n), jnp.float32),
            pltpu.SemaphoreType.DMA((N_CHUNK,)),
            pltpu.SemaphoreType.DMA((N_CHUNK,)),
        ],
        compiler_params=pltpu.CompilerParams(collective_id=0),
    )(A, B)


def _kernel_v3(A, B):
    m, k = A.shape
    _, n = B.shape
    half = m // 2
    qm = m // 4
    NB = 3
    nc = n // NB

    def body(a_ref, b_ref, out_ref, ar_ref, br_ref,
             a_send, a_recv, af_send, af_recv, b_send, b_recv):
        my_x = lax.axis_index("x")
        my_y = lax.axis_index("y")
        p = (1 - my_x, my_y)
        q = (my_x, 1 - my_y)

        barrier_sem = pltpu.get_barrier_semaphore()
        for nbr in (p, q):
            pl.semaphore_signal(barrier_sem, inc=1, device_id=nbr,
                                device_id_type=pl.DeviceIdType.MESH)
        pl.semaphore_wait(barrier_sem, 2)

        def a_direct_rdma(row0, c):
            sl = slice(row0 + c * qm, row0 + (c + 1) * qm)
            return pltpu.make_async_remote_copy(
                src_ref=a_ref.at[sl, :], dst_ref=ar_ref.at[sl, :],
                send_sem=a_send.at[c], recv_sem=a_recv.at[c],
                device_id=p, device_id_type=pl.DeviceIdType.MESH)

        def a_fwd_rdma(row0, c):
            sl = slice(row0 + c * qm, row0 + (c + 1) * qm)
            return pltpu.make_async_remote_copy(
                src_ref=ar_ref.at[sl, :], dst_ref=ar_ref.at[sl, :],
                send_sem=af_send.at[c], recv_sem=af_recv.at[c],
                device_id=q, device_id_type=pl.DeviceIdType.MESH)

        @pl.when(my_y == 0)
        def _():
            for c in range(2):
                a_direct_rdma(0, c).start()

        @pl.when(my_y == 1)
        def _():
            for c in range(2):
                a_direct_rdma(half, c).start()

        b_rdmas = []
        for j in range(NB):
            slc = slice(j * nc, (j + 1) * nc)
            r = pltpu.make_async_remote_copy(
                src_ref=b_ref.at[:, slc], dst_ref=br_ref.at[:, slc],
                send_sem=b_send.at[j], recv_sem=b_recv.at[j],
                device_id=p, device_id_type=pl.DeviceIdType.MESH)
            r.start()
            b_rdmas.append(r)

        out_ref[...] = jnp.dot(a_ref[...], b_ref[...],
                               preferred_element_type=jnp.float32)

        for c in range(2):
            a_direct_rdma(0, c).wait_recv()

            @pl.when(my_y == 0)
            def _(c=c):
                a_fwd_rdma(0, c).start()

            @pl.when(my_y == 1)
            def _(c=c):
                a_fwd_rdma(half, c).start()

        for c in range(2):
            a_fwd_rdma(0, c).wait_recv()

        for j in range(NB):
            slc = slice(j * nc, (j + 1) * nc)
            b_rdmas[j].wait_recv()
            out_ref[:, slc] = out_ref[:, slc] + jnp.dot(
                ar_ref[...], br_ref[:, slc],
                preferred_element_type=jnp.float32)

        for c in range(2):
            a_direct_rdma(0, c).wait_send()
            a_fwd_rdma(0, c).wait_send()
        for r in b_rdmas:
            r.wait_send()

    return pl.pallas_call(
        body,
        out_shape=jax.ShapeDtypeStruct((m, n), jnp.float32),
        in_specs=[
            pl.BlockSpec(memory_space=pltpu.VMEM),
            pl.BlockSpec(memory_space=pltpu.VMEM),
        ],
        out_specs=pl.BlockSpec(memory_space=pltpu.VMEM),
        scratch_shapes=[
            pltpu.VMEM((m, k), jnp.float32),
            pltpu.VMEM((k, n), jnp.float32),
            pltpu.SemaphoreType.DMA((2,)),
            pltpu.SemaphoreType.DMA((2,)),
            pltpu.SemaphoreType.DMA((2,)),
            pltpu.SemaphoreType.DMA((2,)),
            pltpu.SemaphoreType.DMA((NB,)),
            pltpu.SemaphoreType.DMA((NB,)),
        ],
        compiler_params=pltpu.CompilerParams(collective_id=0),
    )(A, B)


def _kernel_v4(A, B, NB=3, local_bf16=False):
    m, k = A.shape
    _, n = B.shape
    half = m // 2
    qm = m // 4
    nc = n // NB

    def body(a_ref, b_ref, out_ref, ab_ref, bb_ref, ar_ref, br_ref,
             a_send, a_recv, af_send, af_recv, b_send, b_recv):
        my_x = lax.axis_index("x")
        my_y = lax.axis_index("y")
        p = (1 - my_x, my_y)
        q = (my_x, 1 - my_y)

        ab_ref[...] = a_ref[...].astype(jnp.bfloat16)
        bb_ref[...] = b_ref[...].astype(jnp.bfloat16)

        barrier_sem = pltpu.get_barrier_semaphore()
        for nbr in (p, q):
            pl.semaphore_signal(barrier_sem, inc=1, device_id=nbr,
                                device_id_type=pl.DeviceIdType.MESH)
        pl.semaphore_wait(barrier_sem, 2)

        def a_direct_rdma(row0, c):
            sl = slice(row0 + c * qm, row0 + (c + 1) * qm)
            return pltpu.make_async_remote_copy(
                src_ref=ab_ref.at[sl, :], dst_ref=ar_ref.at[sl, :],
                send_sem=a_send.at[c], recv_sem=a_recv.at[c],
                device_id=p, device_id_type=pl.DeviceIdType.MESH)

        def a_fwd_rdma(row0, c):
            sl = slice(row0 + c * qm, row0 + (c + 1) * qm)
            return pltpu.make_async_remote_copy(
                src_ref=ar_ref.at[sl, :], dst_ref=ar_ref.at[sl, :],
                send_sem=af_send.at[c], recv_sem=af_recv.at[c],
                device_id=q, device_id_type=pl.DeviceIdType.MESH)

        @pl.when(my_y == 0)
        def _():
            for c in range(2):
                a_direct_rdma(0, c).start()

        @pl.when(my_y == 1)
        def _():
            for c in range(2):
                a_direct_rdma(half, c).start()

        b_rdmas = []
        for j in range(NB):
            slc = slice(j * nc, (j + 1) * nc)
            r = pltpu.make_async_remote_copy(
                src_ref=bb_ref.at[:, slc], dst_ref=br_ref.at[:, slc],
                send_sem=b_send.at[j], recv_sem=b_recv.at[j],
                device_id=p, device_id_type=pl.DeviceIdType.MESH)
            r.start()
            b_rdmas.append(r)

        for c in range(2):
            a_direct_rdma(0, c).wait_recv()

            @pl.when(my_y == 0)
            def _(c=c):
                a_fwd_rdma(0, c).start()

            @pl.when(my_y == 1)
            def _(c=c):
                a_fwd_rdma(half, c).start()

        if local_bf16:
            out_ref[...] = jnp.dot(ab_ref[...], bb_ref[...],
                                   preferred_element_type=jnp.float32)
        else:
            out_ref[...] = jnp.dot(a_ref[...], b_ref[...],
                                   preferred_element_type=jnp.float32)

        for c in range(2):
            a_fwd_rdma(0, c).wait_recv()

        for j in range(NB):
            slc = slice(j * nc, (j + 1) * nc)
            b_rdmas[j].wait_recv()
            out_ref[:, slc] = out_ref[:, slc] + jnp.dot(
                ar_ref[...], br_ref[:, slc],
                preferred_element_type=jnp.float32)

        for c in range(2):
            a_direct_rdma(0, c).wait_send()
            a_fwd_rdma(0, c).wait_send()
        for r in b_rdmas:
            r.wait_send()

    return pl.pallas_call(
        body,
        out_shape=jax.ShapeDtypeStruct((m, n), jnp.float32),
        in_specs=[
            pl.BlockSpec(memory_space=pltpu.VMEM),
            pl.BlockSpec(memory_space=pltpu.VMEM),
        ],
        out_specs=pl.BlockSpec(memory_space=pltpu.VMEM),
        scratch_shapes=[
            pltpu.VMEM((m, k), jnp.bfloat16),
            pltpu.VMEM((k, n), jnp.bfloat16),
            pltpu.VMEM((m, k), jnp.bfloat16),
            pltpu.VMEM((k, n), jnp.bfloat16),
            pltpu.SemaphoreType.DMA((2,)),
            pltpu.SemaphoreType.DMA((2,)),
            pltpu.SemaphoreType.DMA((2,)),
            pltpu.SemaphoreType.DMA((2,)),
            pltpu.SemaphoreType.DMA((NB,)),
            pltpu.SemaphoreType.DMA((NB,)),
        ],
        compiler_params=pltpu.CompilerParams(collective_id=0),
    )(A, B)


def _kernel_v5(A, B):
    return _kernel_v4(A, B, NB=6, local_bf16=True)


def _kernel_comm_floor(A, B):
    m, k = A.shape
    _, n = B.shape
    half = m // 2
    qm = m // 4
    NB = 6
    nc = n // NB

    def body(a_ref, b_ref, out_ref, ab_ref, bb_ref, ar_ref, br_ref,
             a_send, a_recv, af_send, af_recv, b_send, b_recv):
        my_x = lax.axis_index("x")
        my_y = lax.axis_index("y")
        p = (1 - my_x, my_y)
        q = (my_x, 1 - my_y)

        ab_ref[...] = a_ref[...].astype(jnp.bfloat16)
        bb_ref[...] = b_ref[...].astype(jnp.bfloat16)

        barrier_sem = pltpu.get_barrier_semaphore()
        for nbr in (p, q):
            pl.semaphore_signal(barrier_sem, inc=1, device_id=nbr,
                                device_id_type=pl.DeviceIdType.MESH)
        pl.semaphore_wait(barrier_sem, 2)

        def a_direct_rdma(row0, c):
            sl = slice(row0 + c * qm, row0 + (c + 1) * qm)
            return pltpu.make_async_remote_copy(
                src_ref=ab_ref.at[sl, :], dst_ref=ar_ref.at[sl, :],
                send_sem=a_send.at[c], recv_sem=a_recv.at[c],
                device_id=p, device_id_type=pl.DeviceIdType.MESH)

        def a_fwd_rdma(row0, c):
            sl = slice(row0 + c * qm, row0 + (c + 1) * qm)
            return pltpu.make_async_remote_copy(
                src_ref=ar_ref.at[sl, :], dst_ref=ar_ref.at[sl, :],
                send_sem=af_send.at[c], recv_sem=af_recv.at[c],
                device_id=q, device_id_type=pl.DeviceIdType.MESH)

        @pl.when(my_y == 0)
        def _():
            for c in range(2):
                a_direct_rdma(0, c).start()

        @pl.when(my_y == 1)
        def _():
            for c in range(2):
                a_direct_rdma(half, c).start()

        b_rdmas = []
        for j in range(NB):
            slc = slice(j * nc, (j + 1) * nc)
            r = pltpu.make_async_remote_copy(
                src_ref=bb_ref.at[:, slc], dst_ref=br_ref.at[:, slc],
                send_sem=b_send.at[j], recv_sem=b_recv.at[j],
                device_id=p, device_id_type=pl.DeviceIdType.MESH)
            r.start()
            b_rdmas.append(r)

        for c in range(2):
            a_direct_rdma(0, c).wait_recv()

            @pl.when(my_y == 0)
            def _(c=c):
                a_fwd_rdma(0, c).start()

            @pl.when(my_y == 1)
            def _(c=c):
                a_fwd_rdma(half, c).start()

        out_ref[...] = jnp.zeros_like(out_ref)

        for c in range(2):
            a_fwd_rdma(0, c).wait_recv()

        for j in range(NB):
            b_rdmas[j].wait_recv()

        out_ref[0:8, 0:128] = ar_ref[0:8, 0:128].astype(jnp.float32)
        out_ref[8:16, 0:128] = br_ref[0:8, 0:128].astype(jnp.float32)

        for c in range(2):
            a_direct_rdma(0, c).wait_send()
            a_fwd_rdma(0, c).wait_send()
        for r in b_rdmas:
            r.wait_send()

    return pl.pallas_call(
        body,
        out_shape=jax.ShapeDtypeStruct((m, n), jnp.float32),
        in_specs=[
            pl.BlockSpec(memory_space=pltpu.VMEM),
            pl.BlockSpec(memory_space=pltpu.VMEM),
        ],
        out_specs=pl.BlockSpec(memory_space=pltpu.VMEM),
        scratch_shapes=[
            pltpu.VMEM((m, k), jnp.bfloat16),
            pltpu.VMEM((k, n), jnp.bfloat16),
            pltpu.VMEM((m, k), jnp.bfloat16),
            pltpu.VMEM((k, n), jnp.bfloat16),
            pltpu.SemaphoreType.DMA((2,)),
            pltpu.SemaphoreType.DMA((2,)),
            pltpu.SemaphoreType.DMA((2,)),
            pltpu.SemaphoreType.DMA((2,)),
            pltpu.SemaphoreType.DMA((NB,)),
            pltpu.SemaphoreType.DMA((NB,)),
        ],
        compiler_params=pltpu.CompilerParams(collective_id=0),
    )(A, B)


kernel = {
    "compute_only": _kernel_compute_only,
    "rdma_only": _kernel_rdma_only,
    "full": _kernel_full,
    "v3": _kernel_v3,
    "v4": _kernel_v4,
    "v5": _kernel_v5,
    "comm_floor": _kernel_comm_floor,
}[_VARIANT]
